# Initial kernel scaffold; baseline (speedup 1.0000x reference)
import jax
import jax.numpy as jnp
from jax.experimental import pallas as pl


def kernel(s_ids, c_ids, shape_emb, color_emb, pos_emb, W, b):
    raise NotImplementedError("write your pallas kernel here")


# speedup vs baseline: 15.6327x; 15.6327x over previous
import functools

import jax
import jax.numpy as jnp
from jax import lax
from jax.experimental import pallas as pl
from jax.experimental.pallas import tpu as pltpu
from jax.experimental.pallas import tpu_sc as plsc

_B = 16384
_L = 200
_D = 64
_NC = 2
_NS = 16
_NW = _NC * _NS
_BLK = 64
_NBLK = _B // (_NW * _BLK)
_JCH = 2
_LCH = _L // _JCH


def _sc_body(s3, c3, semb, cemb, out_hbm, sidx, cidx, rbuf, obuf, sem):
    wid = lax.axis_index("s") * _NC + lax.axis_index("c")

    def blk_body(blk, carry):
        g = wid * _NBLK + blk
        pltpu.sync_copy(s3.at[g], sidx)
        pltpu.sync_copy(c3.at[g], cidx)

        def row_body(r, rcarry):
            cps = []
            for j in range(_JCH):
                cps.append(pltpu.async_copy(semb.at[sidx.at[r, j]],
                                            rbuf.at[0, j], sem))
                cps.append(pltpu.async_copy(cemb.at[cidx.at[r, j]],
                                            rbuf.at[1, j], sem))
            for cp in cps:
                cp.wait()

            zero = jnp.zeros((16,), jnp.float32)

            def l_body(l, acc):
                acc = list(acc)
                for t in range(2):
                    for j in range(_JCH):
                        for d in range(4):
                            acc[d] = acc[d] + rbuf[t, j, l, pl.ds(d * 16, 16)]
                return tuple(acc)

            acc = lax.fori_loop(0, _LCH, l_body, (zero, zero, zero, zero))
            for d in range(4):
                obuf[r, pl.ds(d * 16, 16)] = acc[d]
            return rcarry

        lax.fori_loop(0, _BLK, row_body, 0)
        pltpu.sync_copy(obuf, out_hbm.at[pl.ds(g * _BLK, _BLK)])
        return carry

    lax.fori_loop(0, _NBLK, blk_body, 0)


@jax.jit
def _sc_gather_sum(s3, c3, semb, cemb):
    return pl.kernel(
        _sc_body,
        mesh=plsc.VectorSubcoreMesh(core_axis_name="c", subcore_axis_name="s"),
        compiler_params=pltpu.CompilerParams(use_tc_tiling_on_sc=False),
        out_type=jax.ShapeDtypeStruct((_B, _D), jnp.float32),
        scratch_types=[
            pltpu.VMEM((_BLK, _JCH, _LCH), jnp.int32),
            pltpu.VMEM((_BLK, _JCH, _LCH), jnp.int32),
            pltpu.VMEM((2, _JCH, _LCH, _D), jnp.float32),
            pltpu.VMEM((_BLK, _D), jnp.float32),
            pltpu.SemaphoreType.DMA,
        ],
    )(s3, c3, semb, cemb)


def _tc_body(s_ref, pos_ref, sums_ref, w_ref, b_ref, o_ref):
    maskf = (s_ref[...] != 0).astype(jnp.float32)
    denom = jnp.maximum(jnp.sum(maskf, axis=1, keepdims=True), 1.0)
    poss = lax.dot_general(maskf, pos_ref[0:_L, :],
                           (((1,), (0,)), ((), ())),
                           preferred_element_type=jnp.float32)
    h = (sums_ref[...] + poss) / denom
    o_ref[...] = lax.dot_general(h, w_ref[...],
                                 (((1,), (0,)), ((), ())),
                                 preferred_element_type=jnp.float32) + b_ref[...]


_TC_BT = 1024


@jax.jit
def _tc_head(s32, pos_emb, sums, w_p, b_p):
    grid = (_B // _TC_BT,)
    return pl.pallas_call(
        _tc_body,
        grid=grid,
        in_specs=[
            pl.BlockSpec((_TC_BT, _L), lambda i: (i, 0)),
            pl.BlockSpec((256, _D), lambda i: (0, 0)),
            pl.BlockSpec((_TC_BT, _D), lambda i: (i, 0)),
            pl.BlockSpec((_D, 128), lambda i: (0, 0)),
            pl.BlockSpec((1, 128), lambda i: (0, 0)),
        ],
        out_specs=pl.BlockSpec((_TC_BT, 128), lambda i: (i, 0)),
        out_shape=jax.ShapeDtypeStruct((_B, 128), jnp.float32),
    )(s32, pos_emb, sums, w_p, b_p)


def kernel(s_ids, c_ids, shape_emb, color_emb, pos_emb, W, b):
    s32 = s_ids.astype(jnp.int32)
    c32 = jnp.where(s32 == 0, 0, c_ids.astype(jnp.int32))
    s3 = s32.reshape(_NW * _NBLK, _BLK, _JCH, _LCH)
    c3 = c32.reshape(_NW * _NBLK, _BLK, _JCH, _LCH)
    sums = _sc_gather_sum(s3, c3, shape_emb, color_emb)
    w_p = jnp.zeros((_D, 128), jnp.float32).at[:, :2].set(W)
    b_p = jnp.zeros((1, 128), jnp.float32).at[0, :2].set(b)
    outp = _tc_head(s32, pos_emb, sums, w_p, b_p)
    return outp[:, :2]


# speedup vs baseline: 23.6161x; 1.5107x over previous
import functools

import jax
import jax.numpy as jnp
from jax import lax
from jax.experimental import pallas as pl
from jax.experimental.pallas import tpu as pltpu
from jax.experimental.pallas import tpu_sc as plsc

_B = 16384
_L = 200
_D = 64
_NC = 2
_NS = 16
_NW = _NC * _NS
_BLK = 64
_NBLK = _B // (_NW * _BLK)
_JCH = 2
_LCH = _L // _JCH


def _sc_body(s3, c3, semb, cemb, out_hbm, sidx, cidx, rbuf, obuf,
             sem0, sem1, sem2, sem3):
    wid = lax.axis_index("s") * _NC + lax.axis_index("c")
    sems = (sem0, sem1, sem2, sem3)

    def _copy(r, q, slot):
        t, j = divmod(q, 2)
        table = semb if t == 0 else cemb
        idx = sidx if t == 0 else cidx
        return pltpu.make_async_copy(table.at[idx.at[r, j]],
                                     rbuf.at[slot], sems[slot])

    def blk_body(blk, carry):
        g = wid * _NBLK + blk
        pltpu.sync_copy(s3.at[g], sidx)
        pltpu.sync_copy(c3.at[g], cidx)

        for q in range(3):
            _copy(0, q, q).start()

        def row_body(r, rcarry):
            zero = jnp.zeros((16,), jnp.float32)
            acc = (zero, zero, zero, zero)
            for q in range(4):
                _copy(r, q, q).wait()
                if q == 0:
                    _copy(r, 3, 3).start()
                else:
                    @pl.when(r + 1 < _BLK)
                    def _():
                        _copy(r + 1, q - 1, q - 1).start()

                def l_body(l, a, _q=q):
                    return tuple(
                        a[d] + rbuf[_q, l, pl.ds(d * 16, 16)] for d in range(4)
                    )

                acc = lax.fori_loop(0, _LCH, l_body, acc)
            for d in range(4):
                obuf[r, pl.ds(d * 16, 16)] = acc[d]
            return rcarry

        lax.fori_loop(0, _BLK, row_body, 0)
        pltpu.sync_copy(obuf, out_hbm.at[pl.ds(g * _BLK, _BLK)])
        return carry

    lax.fori_loop(0, _NBLK, blk_body, 0)


@jax.jit
def _sc_gather_sum(s3, c3, semb, cemb):
    return pl.kernel(
        _sc_body,
        mesh=plsc.VectorSubcoreMesh(core_axis_name="c", subcore_axis_name="s"),
        compiler_params=pltpu.CompilerParams(use_tc_tiling_on_sc=False),
        out_type=jax.ShapeDtypeStruct((_B, _D), jnp.float32),
        scratch_types=[
            pltpu.VMEM((_BLK, _JCH, _LCH), jnp.int32),
            pltpu.VMEM((_BLK, _JCH, _LCH), jnp.int32),
            pltpu.VMEM((4, _LCH, _D), jnp.float32),
            pltpu.VMEM((_BLK, _D), jnp.float32),
            pltpu.SemaphoreType.DMA,
            pltpu.SemaphoreType.DMA,
            pltpu.SemaphoreType.DMA,
            pltpu.SemaphoreType.DMA,
        ],
    )(s3, c3, semb, cemb)


def _tc_body(s_ref, pos_ref, sums_ref, w_ref, b_ref, o_ref):
    maskf = (s_ref[...] != 0).astype(jnp.float32)
    denom = jnp.maximum(jnp.sum(maskf, axis=1, keepdims=True), 1.0)
    poss = lax.dot_general(maskf, pos_ref[0:_L, :],
                           (((1,), (0,)), ((), ())),
                           preferred_element_type=jnp.float32)
    h = (sums_ref[...] + poss) / denom
    o_ref[...] = lax.dot_general(h, w_ref[...],
                                 (((1,), (0,)), ((), ())),
                                 preferred_element_type=jnp.float32) + b_ref[...]


_TC_BT = 1024


@jax.jit
def _tc_head(s32, pos_emb, sums, w_p, b_p):
    grid = (_B // _TC_BT,)
    return pl.pallas_call(
        _tc_body,
        grid=grid,
        in_specs=[
            pl.BlockSpec((_TC_BT, _L), lambda i: (i, 0)),
            pl.BlockSpec((256, _D), lambda i: (0, 0)),
            pl.BlockSpec((_TC_BT, _D), lambda i: (i, 0)),
            pl.BlockSpec((_D, 128), lambda i: (0, 0)),
            pl.BlockSpec((1, 128), lambda i: (0, 0)),
        ],
        out_specs=pl.BlockSpec((_TC_BT, 128), lambda i: (i, 0)),
        out_shape=jax.ShapeDtypeStruct((_B, 128), jnp.float32),
    )(s32, pos_emb, sums, w_p, b_p)


def kernel(s_ids, c_ids, shape_emb, color_emb, pos_emb, W, b):
    s32 = s_ids.astype(jnp.int32)
    c32 = jnp.where(s32 == 0, 0, c_ids.astype(jnp.int32))
    s3 = s32.reshape(_NW * _NBLK, _BLK, _JCH, _LCH)
    c3 = c32.reshape(_NW * _NBLK, _BLK, _JCH, _LCH)
    sums = _sc_gather_sum(s3, c3, shape_emb, color_emb)
    w_p = jnp.zeros((_D, 128), jnp.float32).at[:, :2].set(W)
    b_p = jnp.zeros((1, 128), jnp.float32).at[0, :2].set(b)
    outp = _tc_head(s32, pos_emb, sums, w_p, b_p)
    return outp[:, :2]


# speedup vs baseline: 27.9540x; 1.1837x over previous
import functools

import jax
import jax.numpy as jnp
from jax import lax
from jax.experimental import pallas as pl
from jax.experimental.pallas import tpu as pltpu
from jax.experimental.pallas import tpu_sc as plsc

_B = 16384
_L = 200
_D = 64
_NC = 2
_NS = 16
_NW = _NC * _NS
_BLK = 64
_NBLK = _B // (_NW * _BLK)
_JCH = 2
_LCH = _L // _JCH


_CHUNKS = ((0, 128), (128, 72))


def _sc_body(s_hbm, c_hbm, semb, cemb, out_hbm, sidx, cidx, rbuf, obuf,
             sem0, sem1, sem2, sem3):
    wid = lax.axis_index("s") * _NC + lax.axis_index("c")
    sems = (sem0, sem1, sem2, sem3)

    def _copy(r, q, slot):
        t, j = divmod(q, 2)
        table = semb if t == 0 else cemb
        idx = sidx if t == 0 else cidx
        off, ln = _CHUNKS[j]
        return pltpu.make_async_copy(table.at[idx.at[r, pl.ds(off, ln)]],
                                     rbuf.at[slot, pl.ds(0, ln)], sems[slot])

    def blk_body(blk, carry):
        g = wid * _NBLK + blk
        base = g * _BLK
        pltpu.sync_copy(s_hbm.at[pl.ds(base, _BLK)], sidx)
        pltpu.sync_copy(c_hbm.at[pl.ds(base, _BLK)], cidx)

        def mask_body(r2, mcarry):
            for k in range(13):
                off = _L - 16 if k == 12 else k * 16
                sv = sidx[r2, pl.ds(off, 16)]
                cv = cidx[r2, pl.ds(off, 16)]
                cidx[r2, pl.ds(off, 16)] = jnp.where(sv == 0, 0, cv)
            return mcarry

        lax.fori_loop(0, _BLK, mask_body, 0)

        for q in range(3):
            _copy(0, q, q).start()

        def row_body(r, rcarry):
            zero = jnp.zeros((16,), jnp.float32)
            acc = (zero, zero, zero, zero)
            for q in range(4):
                _copy(r, q, q).wait()
                if q == 0:
                    _copy(r, 3, 3).start()
                else:
                    @pl.when(r + 1 < _BLK)
                    def _():
                        _copy(r + 1, q - 1, q - 1).start()

                def l_body(l, a, _q=q):
                    return tuple(
                        a[d] + rbuf[_q, l, pl.ds(d * 16, 16)] for d in range(4)
                    )

                acc = lax.fori_loop(0, _CHUNKS[q % 2][1], l_body, acc,
                                    unroll=4)
            for d in range(4):
                obuf[r, pl.ds(d * 16, 16)] = acc[d]
            return rcarry

        lax.fori_loop(0, _BLK, row_body, 0)
        pltpu.sync_copy(obuf, out_hbm.at[pl.ds(base, _BLK)])
        return carry

    lax.fori_loop(0, _NBLK, blk_body, 0)


@jax.jit
def _sc_gather_sum(s32, c32, semb, cemb):
    return pl.kernel(
        _sc_body,
        mesh=plsc.VectorSubcoreMesh(core_axis_name="c", subcore_axis_name="s"),
        compiler_params=pltpu.CompilerParams(use_tc_tiling_on_sc=False),
        out_type=jax.ShapeDtypeStruct((_B, _D), jnp.float32),
        scratch_types=[
            pltpu.VMEM((_BLK, _L), jnp.int32),
            pltpu.VMEM((_BLK, _L), jnp.int32),
            pltpu.VMEM((4, 128, _D), jnp.float32),
            pltpu.VMEM((_BLK, _D), jnp.float32),
            pltpu.SemaphoreType.DMA,
            pltpu.SemaphoreType.DMA,
            pltpu.SemaphoreType.DMA,
            pltpu.SemaphoreType.DMA,
        ],
    )(s32, c32, semb, cemb)


def _tc_body(s_ref, pos_ref, sums_ref, w_ref, b_ref, o_ref):
    maskf = (s_ref[...] != 0).astype(jnp.float32)
    denom = jnp.maximum(jnp.sum(maskf, axis=1, keepdims=True), 1.0)
    poss = lax.dot_general(maskf, pos_ref[0:_L, :],
                           (((1,), (0,)), ((), ())),
                           preferred_element_type=jnp.float32)
    h = (sums_ref[...] + poss) / denom
    o_ref[...] = lax.dot_general(h, w_ref[...],
                                 (((1,), (0,)), ((), ())),
                                 preferred_element_type=jnp.float32) + b_ref[...]


_TC_BT = 1024


@jax.jit
def _tc_head(s32, pos_emb, sums, w_p, b_p):
    grid = (_B // _TC_BT,)
    return pl.pallas_call(
        _tc_body,
        grid=grid,
        in_specs=[
            pl.BlockSpec((_TC_BT, _L), lambda i: (i, 0)),
            pl.BlockSpec((256, _D), lambda i: (0, 0)),
            pl.BlockSpec((_TC_BT, _D), lambda i: (i, 0)),
            pl.BlockSpec((_D, 128), lambda i: (0, 0)),
            pl.BlockSpec((1, 128), lambda i: (0, 0)),
        ],
        out_specs=pl.BlockSpec((_TC_BT, 128), lambda i: (i, 0)),
        out_shape=jax.ShapeDtypeStruct((_B, 128), jnp.float32),
    )(s32, pos_emb, sums, w_p, b_p)


def kernel(s_ids, c_ids, shape_emb, color_emb, pos_emb, W, b):
    s32 = s_ids.astype(jnp.int32)
    c32 = c_ids.astype(jnp.int32)
    sums = _sc_gather_sum(s32, c32, shape_emb, color_emb)
    w_p = jnp.zeros((_D, 128), jnp.float32).at[:, :2].set(W)
    b_p = jnp.zeros((1, 128), jnp.float32).at[0, :2].set(b)
    outp = _tc_head(s32, pos_emb, sums, w_p, b_p)
    return outp[:, :2]
